# Initial kernel scaffold; baseline (speedup 1.0000x reference)
import jax
import jax.numpy as jnp
from jax.experimental import pallas as pl


def kernel(boxes, scores):
    raise NotImplementedError("write your pallas kernel here")


# speedup vs baseline: 43.1827x; 43.1827x over previous
import jax
import jax.numpy as jnp
from jax.experimental import pallas as pl

_LANES = 128
_THR = 0.5
_EPS = 0.01


def _nms_body(x1_ref, y1_ref, x2_ref, y2_ref, kept_ref):
    nrows = x1_ref.shape[0]
    kept_ref[...] = jnp.ones_like(kept_ref)

    isub = jax.lax.broadcasted_iota(jnp.int32, (_LANES, _LANES), 0)
    jlan = jax.lax.broadcasted_iota(jnp.int32, (_LANES, _LANES), 1)
    lower = isub < jlan
    upper = isub > jlan

    def block_step(b, _):
        x1r = x1_ref[pl.ds(b, 1), :]
        y1r = y1_ref[pl.ds(b, 1), :]
        x2r = x2_ref[pl.ds(b, 1), :]
        y2r = y2_ref[pl.ds(b, 1), :]
        arr = (x2r - x1r + _EPS) * (y2r - y1r + _EPS)
        x1c = jnp.transpose(x1r)
        y1c = jnp.transpose(y1r)
        x2c = jnp.transpose(x2r)
        y2c = jnp.transpose(y2r)
        arc = jnp.transpose(arr)

        xx1 = jnp.maximum(x1c, x1r)
        yy1 = jnp.maximum(y1c, y1r)
        xx2 = jnp.maximum(x2c, x2r)
        yy2 = jnp.maximum(y2c, y2r)
        w = jnp.maximum(xx2 - xx1 + _EPS, 0.0)
        h = jnp.maximum(yy2 - yy1 + _EPS, 0.0)
        inter = w * h
        iou = inter / (arc + arr - inter)
        hit = iou > _THR
        mr = hit & lower
        mc = hit & upper

        pre_row = kept_ref[pl.ds(b, 1), :]
        pre_col = jnp.transpose(pre_row)

        def resolve(_, kc):
            k_row, k_col = kc
            sup_r = jnp.max(jnp.where(mr, k_col, 0.0), axis=0, keepdims=True)
            k_row = pre_row * (1.0 - sup_r)
            sup_c = jnp.max(jnp.where(mc, k_row, 0.0), axis=1, keepdims=True)
            k_col = pre_col * (1.0 - sup_c)
            return (k_row, k_col)

        k_row, k_col = jax.lax.fori_loop(
            0, _LANES // 2, resolve, (pre_row, pre_col))
        kept_ref[pl.ds(b, 1), :] = k_row

        def tail(c, _):
            cx1 = x1_ref[pl.ds(c, 1), :]
            cy1 = y1_ref[pl.ds(c, 1), :]
            cx2 = x2_ref[pl.ds(c, 1), :]
            cy2 = y2_ref[pl.ds(c, 1), :]
            car = (cx2 - cx1 + _EPS) * (cy2 - cy1 + _EPS)
            txx1 = jnp.maximum(x1c, cx1)
            tyy1 = jnp.maximum(y1c, cy1)
            txx2 = jnp.maximum(x2c, cx2)
            tyy2 = jnp.maximum(y2c, cy2)
            tw = jnp.maximum(txx2 - txx1 + _EPS, 0.0)
            th = jnp.maximum(tyy2 - tyy1 + _EPS, 0.0)
            tinter = tw * th
            tiou = tinter / (arc + car - tinter)
            sup = jnp.max(jnp.where(tiou > _THR, k_col, 0.0),
                          axis=0, keepdims=True)
            kept_ref[pl.ds(c, 1), :] = kept_ref[pl.ds(c, 1), :] * (1.0 - sup)
            return 0

        jax.lax.fori_loop(b + 1, nrows, tail, 0)
        return 0

    jax.lax.fori_loop(0, nrows, block_step, 0)


def _nms_call(x1, y1, x2, y2, interpret=False):
    return pl.pallas_call(
        _nms_body,
        out_shape=jax.ShapeDtypeStruct(x1.shape, jnp.float32),
        interpret=interpret,
    )(x1, y1, x2, y2)


def kernel(boxes, scores):
    n = boxes.shape[0]
    order = jnp.argsort(-scores)
    sb = boxes[order]
    nrows = (n + _LANES - 1) // _LANES
    pad = nrows * _LANES - n
    cols = []
    for k in range(4):
        cols.append(jnp.pad(sb[:, k], (0, pad)).reshape(nrows, _LANES))
    kept = _nms_call(*cols)
    keep_sorted = kept.reshape(-1)[:n] > 0.5
    keep = jnp.zeros((n,), bool).at[order].set(keep_sorted)
    masked = boxes * keep[:, None].astype(boxes.dtype)
    return (masked, keep)


# speedup vs baseline: 49.9743x; 1.1573x over previous
import jax
import jax.numpy as jnp
from jax.experimental import pallas as pl
from jax.experimental.pallas import tpu as pltpu

_LANES = 128
_CR = 8
_THR = 0.5
_EPS = 0.01


def _nms_body(x1_ref, y1_ref, x2_ref, y2_ref, kept_ref, ar_ref):
    nrows_pad, _ = x1_ref.shape
    ngroups = nrows_pad // _CR
    kept_ref[...] = jnp.ones_like(kept_ref)
    ar_ref[...] = ((x2_ref[...] - x1_ref[...] + _EPS)
                   * (y2_ref[...] - y1_ref[...] + _EPS))

    isub = jax.lax.broadcasted_iota(jnp.int32, (_LANES, _LANES), 0)
    jlan = jax.lax.broadcasted_iota(jnp.int32, (_LANES, _LANES), 1)
    lower = isub < jlan
    upper = isub > jlan
    giota = jax.lax.broadcasted_iota(jnp.int32, (_CR, 1), 0)

    def block_step(b, _):
        x1r = x1_ref[pl.ds(b, 1), :]
        y1r = y1_ref[pl.ds(b, 1), :]
        x2r = x2_ref[pl.ds(b, 1), :]
        y2r = y2_ref[pl.ds(b, 1), :]
        arr = ar_ref[pl.ds(b, 1), :]
        x1c = jnp.transpose(x1r)
        y1c = jnp.transpose(y1r)
        x2c = jnp.transpose(x2r)
        y2c = jnp.transpose(y2r)
        arc = jnp.transpose(arr)

        xx1 = jnp.maximum(x1c, x1r)
        yy1 = jnp.maximum(y1c, y1r)
        xx2 = jnp.maximum(x2c, x2r)
        yy2 = jnp.maximum(y2c, y2r)
        w = jnp.maximum(xx2 - xx1 + _EPS, 0.0)
        h = jnp.maximum(yy2 - yy1 + _EPS, 0.0)
        inter = w * h
        iou = inter / (arc + arr - inter)
        hit = iou > _THR
        mr = hit & lower
        mc = hit & upper

        pre_row = kept_ref[pl.ds(b, 1), :]
        pre_col = jnp.transpose(pre_row)

        def resolve(_, kc):
            k_row, k_col = kc
            sup_r = jnp.max(jnp.where(mr, k_col, 0.0), axis=0, keepdims=True)
            k_row = pre_row * (1.0 - sup_r)
            sup_c = jnp.max(jnp.where(mc, k_row, 0.0), axis=1, keepdims=True)
            k_col = pre_col * (1.0 - sup_c)
            return (k_row, k_col)

        k_row, k_col = jax.lax.fori_loop(
            0, _LANES // 2, resolve, (pre_row, pre_col))
        kept_ref[pl.ds(b, 1), :] = k_row

        alive = k_col > 0.0
        sx1 = jnp.where(alive, x1c, 9.0).reshape(1, _LANES, 1)
        sy1 = jnp.where(alive, y1c, 9.0).reshape(1, _LANES, 1)
        sx2 = jnp.where(alive, x2c, 6.0).reshape(1, _LANES, 1)
        sy2 = jnp.where(alive, y2c, 6.0).reshape(1, _LANES, 1)
        sar = jnp.where(alive, arc, 2.0).reshape(1, _LANES, 1)

        def tail(g, _):
            c0 = g * _CR
            cx1 = x1_ref[pl.ds(c0, _CR), :].reshape(_CR, 1, _LANES)
            cy1 = y1_ref[pl.ds(c0, _CR), :].reshape(_CR, 1, _LANES)
            cx2 = x2_ref[pl.ds(c0, _CR), :].reshape(_CR, 1, _LANES)
            cy2 = y2_ref[pl.ds(c0, _CR), :].reshape(_CR, 1, _LANES)
            car = ar_ref[pl.ds(c0, _CR), :].reshape(_CR, 1, _LANES)
            txx1 = jnp.maximum(sx1, cx1)
            tyy1 = jnp.maximum(sy1, cy1)
            txx2 = jnp.maximum(sx2, cx2)
            tyy2 = jnp.maximum(sy2, cy2)
            tw = jnp.maximum(txx2 - txx1 + _EPS, 0.0)
            th = jnp.maximum(tyy2 - tyy1 + _EPS, 0.0)
            tinter = tw * th
            tiou = tinter / (sar + car - tinter)
            sup = jnp.any(tiou > _THR, axis=1)
            valid = (c0 + giota) > b
            tile = kept_ref[pl.ds(c0, _CR), :]
            kept_ref[pl.ds(c0, _CR), :] = jnp.where(sup & valid, 0.0, tile)
            return 0

        jax.lax.fori_loop((b + 1) // _CR, ngroups, tail, 0)
        return 0

    jax.lax.fori_loop(0, nrows_pad, block_step, 0)


def _nms_call(x1, y1, x2, y2, interpret=False):
    return pl.pallas_call(
        _nms_body,
        out_shape=jax.ShapeDtypeStruct(x1.shape, jnp.float32),
        scratch_shapes=[pltpu.VMEM(x1.shape, jnp.float32)],
        interpret=interpret,
    )(x1, y1, x2, y2)


def kernel(boxes, scores):
    n = boxes.shape[0]
    order = jnp.argsort(-scores)
    sb = boxes[order]
    nrows = (n + _LANES - 1) // _LANES
    nrows_pad = ((nrows + _CR - 1) // _CR) * _CR
    pad = nrows_pad * _LANES - n
    cols = []
    for k in range(4):
        cols.append(jnp.pad(sb[:, k], (0, pad)).reshape(nrows_pad, _LANES))
    kept = _nms_call(*cols)
    keep_sorted = kept.reshape(-1)[:n] > 0.5
    keep = jnp.zeros((n,), bool).at[order].set(keep_sorted)
    masked = boxes * keep[:, None].astype(boxes.dtype)
    return (masked, keep)


# speedup vs baseline: 142.1429x; 2.8443x over previous
import jax
import jax.numpy as jnp
from jax.experimental import pallas as pl
from jax.experimental.pallas import tpu as pltpu

_LANES = 128
_CR = 8
_THR = 0.5
_EPS = 0.01


def _nms_body(x1_ref, y1_ref, x2_ref, y2_ref, kept_ref, ar_ref):
    nrows_pad, _ = x1_ref.shape
    ngroups = nrows_pad // _CR
    kept_ref[...] = jnp.ones_like(kept_ref)
    ar_ref[...] = ((x2_ref[...] - x1_ref[...] + _EPS)
                   * (y2_ref[...] - y1_ref[...] + _EPS))

    isub = jax.lax.broadcasted_iota(jnp.int32, (_LANES, _LANES), 0)
    jlan = jax.lax.broadcasted_iota(jnp.int32, (_LANES, _LANES), 1)
    lower = isub < jlan
    upper = isub > jlan
    giota = jax.lax.broadcasted_iota(jnp.int32, (_CR, 1), 0)

    def block_step(b, _):
        x1r = x1_ref[pl.ds(b, 1), :]
        y1r = y1_ref[pl.ds(b, 1), :]
        x2r = x2_ref[pl.ds(b, 1), :]
        y2r = y2_ref[pl.ds(b, 1), :]
        arr = ar_ref[pl.ds(b, 1), :]
        x1c = jnp.transpose(x1r)
        y1c = jnp.transpose(y1r)
        x2c = jnp.transpose(x2r)
        y2c = jnp.transpose(y2r)
        arc = jnp.transpose(arr)

        xx1 = jnp.maximum(x1c, x1r)
        yy1 = jnp.maximum(y1c, y1r)
        xx2 = jnp.maximum(x2c, x2r)
        yy2 = jnp.maximum(y2c, y2r)
        w = jnp.maximum(xx2 - xx1 + _EPS, 0.0)
        h = jnp.maximum(yy2 - yy1 + _EPS, 0.0)
        inter = w * h
        iou = inter / (arc + arr - inter)
        hit = iou > _THR
        mr = hit & lower
        mc = hit & upper

        pre_row = kept_ref[pl.ds(b, 1), :]
        pre_col = jnp.transpose(pre_row)

        def resolve_cond(state):
            return state[0]

        def resolve(state):
            _, k_row, k_col = state
            sup_r = jnp.max(jnp.where(mr, k_col, 0.0), axis=0, keepdims=True)
            k_row = pre_row * (1.0 - sup_r)
            sup_c = jnp.max(jnp.where(mc, k_row, 0.0), axis=1, keepdims=True)
            k_col_n = pre_col * (1.0 - sup_c)
            return (jnp.any(k_col_n != k_col), k_row, k_col_n)

        _, k_row, k_col = jax.lax.while_loop(
            resolve_cond, resolve, (True, pre_row, pre_col))
        kept_ref[pl.ds(b, 1), :] = k_row

        alive = k_col > 0.0
        sx1 = jnp.where(alive, x1c, 9.0).reshape(1, _LANES, 1)
        sy1 = jnp.where(alive, y1c, 9.0).reshape(1, _LANES, 1)
        sx2 = jnp.where(alive, x2c, 6.0).reshape(1, _LANES, 1)
        sy2 = jnp.where(alive, y2c, 6.0).reshape(1, _LANES, 1)
        sar = jnp.where(alive, arc, 2.0).reshape(1, _LANES, 1)

        def tail(g, _):
            c0 = g * _CR
            cx1 = x1_ref[pl.ds(c0, _CR), :].reshape(_CR, 1, _LANES)
            cy1 = y1_ref[pl.ds(c0, _CR), :].reshape(_CR, 1, _LANES)
            cx2 = x2_ref[pl.ds(c0, _CR), :].reshape(_CR, 1, _LANES)
            cy2 = y2_ref[pl.ds(c0, _CR), :].reshape(_CR, 1, _LANES)
            car = ar_ref[pl.ds(c0, _CR), :].reshape(_CR, 1, _LANES)
            txx1 = jnp.maximum(sx1, cx1)
            tyy1 = jnp.maximum(sy1, cy1)
            txx2 = jnp.maximum(sx2, cx2)
            tyy2 = jnp.maximum(sy2, cy2)
            tw = jnp.maximum(txx2 - txx1 + _EPS, 0.0)
            th = jnp.maximum(tyy2 - tyy1 + _EPS, 0.0)
            tinter = tw * th
            tiou = tinter / (sar + car - tinter)
            sup = jnp.any(tiou > _THR, axis=1)
            valid = (c0 + giota) > b
            tile = kept_ref[pl.ds(c0, _CR), :]
            kept_ref[pl.ds(c0, _CR), :] = jnp.where(sup & valid, 0.0, tile)
            return 0

        jax.lax.fori_loop((b + 1) // _CR, ngroups, tail, 0)
        return 0

    jax.lax.fori_loop(0, nrows_pad, block_step, 0)


def _nms_call(x1, y1, x2, y2, interpret=False):
    return pl.pallas_call(
        _nms_body,
        out_shape=jax.ShapeDtypeStruct(x1.shape, jnp.float32),
        scratch_shapes=[pltpu.VMEM(x1.shape, jnp.float32)],
        interpret=interpret,
    )(x1, y1, x2, y2)


def kernel(boxes, scores):
    n = boxes.shape[0]
    order = jnp.argsort(-scores)
    sb = boxes[order]
    nrows = (n + _LANES - 1) // _LANES
    nrows_pad = ((nrows + _CR - 1) // _CR) * _CR
    pad = nrows_pad * _LANES - n
    cols = []
    for k in range(4):
        cols.append(jnp.pad(sb[:, k], (0, pad)).reshape(nrows_pad, _LANES))
    kept = _nms_call(*cols)
    keep_sorted = kept.reshape(-1)[:n] > 0.5
    keep = jnp.zeros((n,), bool).at[order].set(keep_sorted)
    masked = boxes * keep[:, None].astype(boxes.dtype)
    return (masked, keep)
